# baseline (speedup 1.0000x reference)
import jax
import jax.numpy as jnp
from jax import lax
from jax.experimental import pallas as pl
from jax.experimental.pallas import tpu as pltpu
from jax.experimental.pallas import tpu_sc as plsc

_P_THR = 0.55
_C_THR = 0.7
_W_FEAT = 1.0
_W_ATTN = 1.0

_T = 512
_D = 512
_NSEG = 256
_NQ = 4
_QT = _T // _NQ
_LSEG = 64
_TRASH = _LSEG
_BIG = 1 << 20
_CH = 32


def _seg_ids(a):
    pred = a > _P_THR
    pred_f = jnp.where(pred, 1.0, 0.0)
    r = lax.broadcasted_iota(jnp.int32, (_T, _T), 0)
    c = lax.broadcasted_iota(jnp.int32, (_T, _T), 1)
    shift = jnp.where(r + 1 == c, 1.0, 0.0)
    triu = jnp.where(r <= c, 1.0, 0.0)
    prev_f = jnp.dot(pred_f, shift, preferred_element_type=jnp.float32)
    start_f = pred_f * (1.0 - prev_f)
    cum = jnp.dot(start_f, triu, preferred_element_type=jnp.float32)
    seg = cum.astype(jnp.int32) - 1
    col = lax.broadcasted_iota(jnp.int32, (1, _T), 1)
    return pred, pred_f, seg, col


def _quarter_firsts(pred, seg, col):
    segm = jnp.where(pred, seg, _BIG)
    gfs = []
    for q in range(_NQ):
        mask_q = (col >= q * _QT) & (col < (q + 1) * _QT)
        gfs.append(jnp.min(jnp.where(mask_q, segm, _BIG)))
    return gfs


def _prep_kernel(attn_ref, idx_ref):
    a = attn_ref[0]
    pred, _, seg, col = _seg_ids(a)
    rep = a > _C_THR
    gfs = _quarter_firsts(pred, seg, col)
    q_of_col = col // _QT
    gf_vec = jnp.full((1, _T), _BIG, jnp.int32)
    for q in range(_NQ):
        gf_vec = jnp.where(q_of_col == q, gfs[q], gf_vec)
    l = seg - gf_vec
    idx_p = jnp.where(pred, l, _TRASH)
    idx_r = jnp.where(rep, l, _TRASH)
    rows = []
    for q in range(_NQ):
        p_sl = lax.slice(idx_p, (0, q * _QT), (1, (q + 1) * _QT))
        r_sl = lax.slice(idx_r, (0, q * _QT), (1, (q + 1) * _QT))
        rows.append(jnp.concatenate([p_sl, r_sl], axis=1))
    idx_ref[0] = jnp.concatenate(rows, axis=0)


def _sc_body(feat_hbm, idx_hbm, out_hbm, acc, idxb, stage):
    c = lax.axis_index("c")
    s = lax.axis_index("s")
    w = c * 16 + s
    b = w // _NQ
    q = w % _NQ

    pltpu.sync_copy(idx_hbm.at[b, q], idxb)

    zero16 = jnp.zeros((16,), jnp.float32)
    nrows = 2 * (_LSEG + 1)
    rep_off = (_LSEG + 1) * _D

    def zrow(i, carry):
        row = acc.at[pl.ds(i * _D, _D)]
        for j in range(_D // 16):
            row[pl.ds(j * 16, 16)] = zero16
        return carry

    lax.fori_loop(0, nrows, zrow, 0)

    def chunk_body(k, carry):
        pltpu.sync_copy(
            feat_hbm.at[b, pl.ds(q * _QT * _D + k * (_CH * _D), _CH * _D)], stage
        )
        for g in range(_CH // 16):
            ipv = idxb[pl.ds(k * _CH + g * 16, 16)]
            irv = idxb[pl.ds(_QT + k * _CH + g * 16, 16)]
            for r2 in range(16):
                r = g * 16 + r2
                ip = ipv[r2]
                ir = irv[r2]
                srow = stage.at[pl.ds(r * _D, _D)]

                @pl.when(ip != _TRASH)
                def _p(ip=ip, srow=srow):
                    prow = acc.at[pl.ds(ip * _D, _D)]
                    for j in range(_D // 16):
                        plsc.addupdate(prow.at[pl.ds(j * 16, 16)], srow[pl.ds(j * 16, 16)])

                @pl.when(ir != _TRASH)
                def _r(ir=ir, srow=srow):
                    rrow = acc.at[pl.ds(rep_off + ir * _D, _D)]
                    for j in range(_D // 16):
                        plsc.addupdate(rrow.at[pl.ds(j * 16, 16)], srow[pl.ds(j * 16, 16)])

        return carry

    lax.fori_loop(0, _QT // _CH, chunk_body, 0)

    half = _LSEG * _D
    pltpu.sync_copy(acc.at[pl.ds(0, half)], out_hbm.at[w, pl.ds(0, half)])
    pltpu.sync_copy(acc.at[pl.ds(rep_off, half)], out_hbm.at[w, pl.ds(half, half)])


def _sc_scatter(feat_flat, idx_w):
    run = pl.kernel(
        _sc_body,
        out_type=jax.ShapeDtypeStruct((32, 2 * _LSEG * _D), jnp.float32),
        mesh=plsc.VectorSubcoreMesh(
            core_axis_name="c", subcore_axis_name="s", num_cores=2, num_subcores=16
        ),
        scratch_types=[
            pltpu.VMEM((2 * (_LSEG + 1) * _D,), jnp.float32),
            pltpu.VMEM((2 * _QT,), jnp.int32),
            pltpu.VMEM((_CH * _D,), jnp.float32),
        ],
    )
    return run(feat_flat, idx_w)


def _finalize_kernel(attn_ref, acc_hbm, out_ref, sacc_ref, accv_ref, sem):
    bidx = pl.program_id(0)
    nb = pl.num_programs(0)

    cp = pltpu.make_async_copy(acc_hbm.at[bidx], accv_ref, sem)
    cp.start()

    @pl.when(bidx == 0)
    def _init():
        sacc_ref[0] = 0.0
        sacc_ref[1] = 0.0
        sacc_ref[2] = 0.0

    a = attn_ref[0]
    pred, pred_f, seg, col = _seg_ids(a)
    rep_f = jnp.where(a > _C_THR, 1.0, 0.0)
    gfs = _quarter_firsts(pred, seg, col)

    row_ids = lax.broadcasted_iota(jnp.int32, (_NSEG, _T), 0)
    m = jnp.where((row_ids == seg) & pred, 1.0, 0.0)
    m_rep = m * rep_f

    counts = jnp.sum(m, axis=1, keepdims=True)
    rep_counts = jnp.sum(m_rep, axis=1, keepdims=True)
    sum_a = jnp.sum(m * a, axis=1, keepdims=True)
    sum_a2 = jnp.sum(m * (a * a), axis=1, keepdims=True)

    valid = counts > 0.0
    counts_safe = jnp.where(valid, counts, 1.0)
    mean_a = sum_a / counts_safe
    var = sum_a2 / counts_safe - mean_a * mean_a
    nprop = jnp.sum(jnp.where(valid, 1.0, 0.0))
    video_loss = jnp.sum(jnp.where(valid, var, 0.0))
    attn_contrib = jnp.where(nprop > 0.0, video_loss / jnp.maximum(nprop, 1.0), 0.0)

    cp.wait()

    r256 = lax.broadcasted_iota(jnp.int32, (_NSEG, _LSEG), 0)
    c64 = lax.broadcasted_iota(jnp.int32, (_NSEG, _LSEG), 1)
    acc_p = jnp.zeros((_NSEG, _D), jnp.float32)
    acc_r = jnp.zeros((_NSEG, _D), jnp.float32)
    for q in range(_NQ):
        mq = jnp.where(r256 == gfs[q] + c64, 1.0, 0.0)
        acc_p = acc_p + jnp.dot(mq, accv_ref[q, 0], preferred_element_type=jnp.float32)
        acc_r = acc_r + jnp.dot(mq, accv_ref[q, 1], preferred_element_type=jnp.float32)

    has_rep = valid & (rep_counts > 0.0)
    rep_safe = jnp.where(has_rep, rep_counts, 1.0)
    diff = acc_p / counts_safe - acc_r / rep_safe
    mse = jnp.sum(diff * diff, axis=1, keepdims=True) / _D
    feat_contrib = jnp.sum(jnp.where(has_rep, mse, 0.0))
    cnt_contrib = jnp.sum(jnp.where(has_rep, 1.0, 0.0))

    sacc_ref[0] += feat_contrib
    sacc_ref[1] += cnt_contrib
    sacc_ref[2] += attn_contrib

    @pl.when(bidx == nb - 1)
    def _fin():
        fls = sacc_ref[0]
        fc = sacc_ref[1]
        feat_loss = jnp.where(fc > 0.0, fls / jnp.maximum(fc, 1.0), fls)
        out_ref[0, 0] = _W_FEAT * feat_loss + _W_ATTN * sacc_ref[2] / nb


def kernel(attn, feat):
    B = attn.shape[0]
    attn3 = attn.reshape(B, 1, _T)

    idx_w = pl.pallas_call(
        _prep_kernel,
        grid=(B,),
        in_specs=[pl.BlockSpec((1, 1, _T), lambda b: (b, 0, 0))],
        out_specs=pl.BlockSpec((1, _NQ, 2 * _QT), lambda b: (b, 0, 0)),
        out_shape=jax.ShapeDtypeStruct((B, _NQ, 2 * _QT), jnp.int32),
    )(attn3)

    feat_flat = feat.reshape(B, _T * _D)
    acc_w = _sc_scatter(feat_flat, idx_w)
    acc5 = acc_w.reshape(B, _NQ, 2, _LSEG, _D)

    out = pl.pallas_call(
        _finalize_kernel,
        grid=(B,),
        in_specs=[
            pl.BlockSpec((1, 1, _T), lambda b: (b, 0, 0)),
            pl.BlockSpec(memory_space=pl.ANY),
        ],
        out_specs=pl.BlockSpec(memory_space=pltpu.SMEM),
        out_shape=jax.ShapeDtypeStruct((1, 1), jnp.float32),
        scratch_shapes=[
            pltpu.SMEM((3,), jnp.float32),
            pltpu.VMEM((_NQ, 2, _LSEG, _D), jnp.float32),
            pltpu.SemaphoreType.DMA,
        ],
    )(attn3, acc5)
    return out[0, 0]


# speedup vs baseline: 1.1662x; 1.1662x over previous
import jax
import jax.numpy as jnp
from jax import lax
from jax.experimental import pallas as pl
from jax.experimental.pallas import tpu as pltpu
from jax.experimental.pallas import tpu_sc as plsc

_P_THR = 0.55
_C_THR = 0.7
_W_FEAT = 1.0
_W_ATTN = 1.0

_T = 512
_D = 512
_NSEG = 256
_NQ = 4
_QT = _T // _NQ
_LSEG = 64
_TRASH = _LSEG
_BIG = 1 << 20
_CH = 32


def _prep_kernel(attn_ref, idx_ref, aux_ref, gfv_ref):
    B = attn_ref.shape[0]
    a = attn_ref[:, 0, :]
    pred = a > _P_THR
    pred_f = jnp.where(pred, 1.0, 0.0)
    r = lax.broadcasted_iota(jnp.int32, (_T, _T), 0)
    c = lax.broadcasted_iota(jnp.int32, (_T, _T), 1)
    shift = jnp.where(r + 1 == c, 1.0, 0.0)
    triu = jnp.where(r <= c, 1.0, 0.0)
    prev_f = jnp.dot(pred_f, shift, preferred_element_type=jnp.float32)
    start_f = pred_f * (1.0 - prev_f)
    cum = jnp.dot(start_f, triu, preferred_element_type=jnp.float32)
    seg = cum.astype(jnp.int32) - 1
    col = lax.broadcasted_iota(jnp.int32, (B, _T), 1)
    rep = a > _C_THR

    segm = jnp.where(pred, seg, _BIG)
    gfs = []
    for q in range(_NQ):
        mask_q = (col >= q * _QT) & (col < (q + 1) * _QT)
        gfs.append(jnp.min(jnp.where(mask_q, segm, _BIG), axis=1, keepdims=True))

    q_of_col = col // _QT
    gf_vec = jnp.full((B, _T), _BIG, jnp.int32)
    for q in range(_NQ):
        gf_vec = jnp.where(q_of_col == q, gfs[q], gf_vec)
    l = seg - gf_vec
    idx_p = jnp.where(pred, l, _TRASH)
    idx_r = jnp.where(rep, l, _TRASH)
    for q in range(_NQ):
        p_sl = lax.slice(idx_p, (0, q * _QT), (B, (q + 1) * _QT))
        r_sl = lax.slice(idx_r, (0, q * _QT), (B, (q + 1) * _QT))
        idx_ref[q] = jnp.concatenate([p_sl, r_sl], axis=1)

    row_ids = lax.broadcasted_iota(jnp.int32, (_NSEG, _T), 0)
    lane = lax.broadcasted_iota(jnp.int32, (_NSEG, 128), 1)
    lane1 = lax.broadcasted_iota(jnp.int32, (1, 128), 1)
    rep_f = jnp.where(rep, 1.0, 0.0)
    for b in range(B):
        seg_b = lax.slice(seg, (b, 0), (b + 1, _T))
        pred_b = lax.slice(pred_f, (b, 0), (b + 1, _T))
        rep_b = lax.slice(rep_f, (b, 0), (b + 1, _T))
        a_b = lax.slice(a, (b, 0), (b + 1, _T))
        m = jnp.where(row_ids == seg_b, 1.0, 0.0) * pred_b
        m_rep = m * rep_b
        counts = jnp.sum(m, axis=1, keepdims=True)
        rep_counts = jnp.sum(m_rep, axis=1, keepdims=True)
        sum_a = jnp.sum(m * a_b, axis=1, keepdims=True)
        sum_a2 = jnp.sum(m * (a_b * a_b), axis=1, keepdims=True)
        valid = counts > 0.0
        counts_safe = jnp.where(valid, counts, 1.0)
        mean_a = sum_a / counts_safe
        var = sum_a2 / counts_safe - mean_a * mean_a
        nprop = jnp.sum(jnp.where(valid, 1.0, 0.0))
        video_loss = jnp.sum(jnp.where(valid, var, 0.0))
        attn_contrib = jnp.where(
            nprop > 0.0, video_loss / jnp.maximum(nprop, 1.0), 0.0
        )
        has_rep = jnp.where(valid & (rep_counts > 0.0), 1.0, 0.0)
        rep_safe = jnp.where(has_rep > 0.0, rep_counts, 1.0)
        aux = (
            jnp.where(lane == 0, counts_safe, 0.0)
            + jnp.where(lane == 1, rep_safe, 0.0)
            + jnp.where(lane == 2, has_rep, 0.0)
        )
        aux_ref[b] = aux
        gfv = jnp.where(lane1 == 4, attn_contrib, 0.0)
        for q in range(_NQ):
            gq = jnp.sum(lax.slice(gfs[q], (b, 0), (b + 1, 1)))
            gfv = jnp.where(lane1 == q, gq.astype(jnp.float32), gfv)
        gfv_ref[b] = gfv


def _sc_body(feat_hbm, idx_hbm, out_hbm, acc, idxb, stage):
    c = lax.axis_index("c")
    s = lax.axis_index("s")
    w = c * 16 + s
    b = w // _NQ
    q = w % _NQ

    pltpu.sync_copy(idx_hbm.at[q, b], idxb)

    zero16 = jnp.zeros((16,), jnp.float32)
    nrows = 2 * (_LSEG + 1)
    rep_off = (_LSEG + 1) * _D

    def zrow(i, carry):
        row = acc.at[pl.ds(i * _D, _D)]
        for j in range(_D // 16):
            row[pl.ds(j * 16, 16)] = zero16
        return carry

    lax.fori_loop(0, nrows, zrow, 0)

    def chunk_body(k, carry):
        pltpu.sync_copy(
            feat_hbm.at[b, pl.ds(q * _QT * _D + k * (_CH * _D), _CH * _D)], stage
        )
        for g in range(_CH // 16):
            ipv = idxb[pl.ds(k * _CH + g * 16, 16)]
            irv = idxb[pl.ds(_QT + k * _CH + g * 16, 16)]
            for r2 in range(16):
                r = g * 16 + r2
                po = ipv[r2] * _D
                ro = rep_off + irv[r2] * _D
                for j in range(_D // 16):
                    v = stage[pl.ds(r * _D + j * 16, 16)]
                    plsc.addupdate(acc.at[pl.ds(po + j * 16, 16)], v)
                    plsc.addupdate(acc.at[pl.ds(ro + j * 16, 16)], v)
        return carry

    lax.fori_loop(0, _QT // _CH, chunk_body, 0)

    half = _LSEG * _D
    pltpu.sync_copy(acc.at[pl.ds(0, half)], out_hbm.at[w, pl.ds(0, half)])
    pltpu.sync_copy(acc.at[pl.ds(rep_off, half)], out_hbm.at[w, pl.ds(half, half)])


def _sc_scatter(feat_flat, idx_w):
    run = pl.kernel(
        _sc_body,
        out_type=jax.ShapeDtypeStruct((32, 2 * _LSEG * _D), jnp.float32),
        mesh=plsc.VectorSubcoreMesh(
            core_axis_name="c", subcore_axis_name="s", num_cores=2, num_subcores=16
        ),
        scratch_types=[
            pltpu.VMEM((2 * (_LSEG + 1) * _D,), jnp.float32),
            pltpu.VMEM((2 * _QT,), jnp.int32),
            pltpu.VMEM((_CH * _D,), jnp.float32),
        ],
    )
    return run(feat_flat, idx_w)


def _finalize_kernel(aux_ref, gfv_ref, acc_hbm, out_ref, sacc_ref, accv_ref, sem):
    bidx = pl.program_id(0)
    nb = pl.num_programs(0)

    cp = pltpu.make_async_copy(acc_hbm.at[bidx], accv_ref, sem)
    cp.start()

    @pl.when(bidx == 0)
    def _init():
        sacc_ref[0] = 0.0
        sacc_ref[1] = 0.0
        sacc_ref[2] = 0.0

    aux = aux_ref[0]
    counts_safe = lax.slice(aux, (0, 0), (_NSEG, 1))
    rep_safe = lax.slice(aux, (0, 1), (_NSEG, 2))
    has_rep = lax.slice(aux, (0, 2), (_NSEG, 3))
    gfv = gfv_ref[0]
    attn_contrib = jnp.sum(lax.slice(gfv, (0, 4), (1, 5)))

    r256 = lax.broadcasted_iota(jnp.int32, (_NSEG, _LSEG), 0)
    c64 = lax.broadcasted_iota(jnp.int32, (_NSEG, _LSEG), 1)
    mqs = []
    for q in range(_NQ):
        gq = jnp.sum(lax.slice(gfv, (0, q), (1, q + 1))).astype(jnp.int32)
        mqs.append(jnp.where(r256 == gq + c64, 1.0, 0.0))

    cp.wait()
    acc_p = jnp.zeros((_NSEG, _D), jnp.float32)
    acc_r = jnp.zeros((_NSEG, _D), jnp.float32)
    for q in range(_NQ):
        acc_p = acc_p + jnp.dot(mqs[q], accv_ref[q, 0], preferred_element_type=jnp.float32)
        acc_r = acc_r + jnp.dot(mqs[q], accv_ref[q, 1], preferred_element_type=jnp.float32)

    diff = acc_p / counts_safe - acc_r / rep_safe
    mse = jnp.sum(diff * diff, axis=1, keepdims=True) / _D
    feat_contrib = jnp.sum(has_rep * mse)
    cnt_contrib = jnp.sum(has_rep)

    sacc_ref[0] += feat_contrib
    sacc_ref[1] += cnt_contrib
    sacc_ref[2] += attn_contrib

    @pl.when(bidx == nb - 1)
    def _fin():
        fls = sacc_ref[0]
        fc = sacc_ref[1]
        feat_loss = jnp.where(fc > 0.0, fls / jnp.maximum(fc, 1.0), fls)
        out_ref[0, 0] = _W_FEAT * feat_loss + _W_ATTN * sacc_ref[2] / nb


def kernel(attn, feat):
    B = attn.shape[0]
    attn3 = attn.reshape(B, 1, _T)

    idx_w, aux, gfv = pl.pallas_call(
        _prep_kernel,
        in_specs=[pl.BlockSpec((B, 1, _T), lambda: (0, 0, 0))],
        out_specs=[
            pl.BlockSpec((_NQ, B, 2 * _QT), lambda: (0, 0, 0)),
            pl.BlockSpec((B, _NSEG, 128), lambda: (0, 0, 0)),
            pl.BlockSpec((B, 1, 128), lambda: (0, 0, 0)),
        ],
        out_shape=[
            jax.ShapeDtypeStruct((_NQ, B, 2 * _QT), jnp.int32),
            jax.ShapeDtypeStruct((B, _NSEG, 128), jnp.float32),
            jax.ShapeDtypeStruct((B, 1, 128), jnp.float32),
        ],
    )(attn3)

    feat_flat = feat.reshape(B, _T * _D)
    acc_w = _sc_scatter(feat_flat, idx_w)
    acc5 = acc_w.reshape(B, _NQ, 2, _LSEG, _D)

    out = pl.pallas_call(
        _finalize_kernel,
        grid=(B,),
        in_specs=[
            pl.BlockSpec((1, _NSEG, 128), lambda b: (b, 0, 0)),
            pl.BlockSpec((1, 1, 128), lambda b: (b, 0, 0)),
            pl.BlockSpec(memory_space=pl.ANY),
        ],
        out_specs=pl.BlockSpec(memory_space=pltpu.SMEM),
        out_shape=jax.ShapeDtypeStruct((1, 1), jnp.float32),
        scratch_shapes=[
            pltpu.SMEM((3,), jnp.float32),
            pltpu.VMEM((_NQ, 2, _LSEG, _D), jnp.float32),
            pltpu.SemaphoreType.DMA,
        ],
    )(aux, gfv, acc5)
    return out[0, 0]


# speedup vs baseline: 1.2948x; 1.1103x over previous
import jax
import jax.numpy as jnp
from jax import lax
from jax.experimental import pallas as pl
from jax.experimental.pallas import tpu as pltpu
from jax.experimental.pallas import tpu_sc as plsc

_P_THR = 0.55
_C_THR = 0.7
_W_FEAT = 1.0
_W_ATTN = 1.0

_T = 512
_D = 512
_NSEG = 256
_NQ = 4
_QT = _T // _NQ
_LSEG = 64
_TRASH = _LSEG
_BIG = 1 << 20
_CH = 32


def _prep_kernel(attn_ref, idx_ref, aux_ref, gfv_ref):
    B = attn_ref.shape[0]
    a = attn_ref[:, 0, :]
    pred = a > _P_THR
    pred_f = jnp.where(pred, 1.0, 0.0)
    r = lax.broadcasted_iota(jnp.int32, (_T, _T), 0)
    c = lax.broadcasted_iota(jnp.int32, (_T, _T), 1)
    shift = jnp.where(r + 1 == c, 1.0, 0.0)
    triu = jnp.where(r <= c, 1.0, 0.0)
    prev_f = jnp.dot(pred_f, shift, preferred_element_type=jnp.float32)
    start_f = pred_f * (1.0 - prev_f)
    cum = jnp.dot(start_f, triu, preferred_element_type=jnp.float32)
    seg = cum.astype(jnp.int32) - 1
    col = lax.broadcasted_iota(jnp.int32, (B, _T), 1)
    rep = a > _C_THR

    segm = jnp.where(pred, seg, _BIG)
    gfs = []
    for q in range(_NQ):
        mask_q = (col >= q * _QT) & (col < (q + 1) * _QT)
        gfs.append(jnp.min(jnp.where(mask_q, segm, _BIG), axis=1, keepdims=True))

    q_of_col = col // _QT
    gf_vec = jnp.full((B, _T), _BIG, jnp.int32)
    for q in range(_NQ):
        gf_vec = jnp.where(q_of_col == q, gfs[q], gf_vec)
    l = seg - gf_vec
    idx_p = jnp.where(pred, l, _TRASH)
    idx_r = jnp.where(rep, l, _TRASH)
    for q in range(_NQ):
        p_sl = lax.slice(idx_p, (0, q * _QT), (B, (q + 1) * _QT))
        r_sl = lax.slice(idx_r, (0, q * _QT), (B, (q + 1) * _QT))
        idx_ref[q] = jnp.concatenate([p_sl, r_sl], axis=1)

    row_ids = lax.broadcasted_iota(jnp.int32, (_NSEG, _T), 0)
    lane = lax.broadcasted_iota(jnp.int32, (_NSEG, 128), 1)
    lane1 = lax.broadcasted_iota(jnp.int32, (1, 128), 1)
    rep_f = jnp.where(rep, 1.0, 0.0)
    for b in range(B):
        seg_b = lax.slice(seg, (b, 0), (b + 1, _T))
        pred_b = lax.slice(pred_f, (b, 0), (b + 1, _T))
        rep_b = lax.slice(rep_f, (b, 0), (b + 1, _T))
        a_b = lax.slice(a, (b, 0), (b + 1, _T))
        m = jnp.where(row_ids == seg_b, 1.0, 0.0) * pred_b
        m_rep = m * rep_b
        counts = jnp.sum(m, axis=1, keepdims=True)
        rep_counts = jnp.sum(m_rep, axis=1, keepdims=True)
        sum_a = jnp.sum(m * a_b, axis=1, keepdims=True)
        sum_a2 = jnp.sum(m * (a_b * a_b), axis=1, keepdims=True)
        valid = counts > 0.0
        counts_safe = jnp.where(valid, counts, 1.0)
        mean_a = sum_a / counts_safe
        var = sum_a2 / counts_safe - mean_a * mean_a
        nprop = jnp.sum(jnp.where(valid, 1.0, 0.0))
        video_loss = jnp.sum(jnp.where(valid, var, 0.0))
        attn_contrib = jnp.where(
            nprop > 0.0, video_loss / jnp.maximum(nprop, 1.0), 0.0
        )
        has_rep = jnp.where(valid & (rep_counts > 0.0), 1.0, 0.0)
        rep_safe = jnp.where(has_rep > 0.0, rep_counts, 1.0)
        aux = (
            jnp.where(lane == 0, counts_safe, 0.0)
            + jnp.where(lane == 1, rep_safe, 0.0)
            + jnp.where(lane == 2, has_rep, 0.0)
        )
        aux_ref[b] = aux
        gfv = jnp.where(lane1 == 4, attn_contrib, 0.0)
        for q in range(_NQ):
            gq = jnp.sum(lax.slice(gfs[q], (b, 0), (b + 1, 1)))
            gfv = jnp.where(lane1 == q, gq.astype(jnp.float32), gfv)
        gfv_ref[b] = gfv


def _sc_body(feat_hbm, idx_hbm, out_hbm, acc, idxb, stage):
    c = lax.axis_index("c")
    s = lax.axis_index("s")
    w = c * 16 + s
    b = w // _NQ
    q = w % _NQ

    pltpu.sync_copy(idx_hbm.at[q, b], idxb)

    zero16 = jnp.zeros((16,), jnp.float32)
    nrows = 2 * (_LSEG + 1)

    def zrow(i, carry):
        m = i // (_LSEG + 1)
        row = i % (_LSEG + 1)
        for j in range(_D // 16):
            acc[m, row, 0, pl.ds(j * 16, 16)] = zero16
        return carry

    lax.fori_loop(0, nrows, zrow, 0)

    def chunk_body(k, carry):
        pltpu.sync_copy(
            feat_hbm.at[b, pl.ds(q * _QT * _D + k * (_CH * _D), _CH * _D)], stage
        )
        for g in range(_CH // 16):
            ipv = idxb[pl.ds(k * _CH + g * 16, 16)]
            irv = idxb[pl.ds(_QT + k * _CH + g * 16, 16)]
            for r2 in range(16):
                r = g * 16 + r2
                ip = ipv[r2]
                ir = irv[r2]
                for j in range(_D // 16):
                    v = stage[pl.ds(r * _D + j * 16, 16)]
                    plsc.addupdate(acc.at[0, ip, 0, pl.ds(j * 16, 16)], v)
                    plsc.addupdate(acc.at[1, ir, 0, pl.ds(j * 16, 16)], v)
        return carry

    lax.fori_loop(0, _QT // _CH, chunk_body, 0)

    pltpu.sync_copy(acc.at[0, pl.ds(0, _LSEG)], out_hbm.at[b, 0, pl.ds(q * _LSEG, _LSEG)])
    pltpu.sync_copy(acc.at[1, pl.ds(0, _LSEG)], out_hbm.at[b, 1, pl.ds(q * _LSEG, _LSEG)])


def _sc_scatter(feat_flat, idx_w):
    run = pl.kernel(
        _sc_body,
        out_type=jax.ShapeDtypeStruct((8, 2, _NQ * _LSEG, 1, _D), jnp.float32),
        mesh=plsc.VectorSubcoreMesh(
            core_axis_name="c", subcore_axis_name="s", num_cores=2, num_subcores=16
        ),
        scratch_types=[
            pltpu.VMEM((2, _LSEG + 1, 1, _D), jnp.float32),
            pltpu.VMEM((2 * _QT,), jnp.int32),
            pltpu.VMEM((_CH * _D,), jnp.float32),
        ],
    )
    return run(feat_flat, idx_w)


def _finalize_kernel(aux_ref, gfv_ref, acc_hbm, out_ref, sacc_ref, accv_ref, sem):
    bidx = pl.program_id(0)
    nb = pl.num_programs(0)

    cp = pltpu.make_async_copy(acc_hbm.at[bidx], accv_ref, sem)
    cp.start()

    @pl.when(bidx == 0)
    def _init():
        sacc_ref[0] = 0.0
        sacc_ref[1] = 0.0
        sacc_ref[2] = 0.0

    aux = aux_ref[0]
    counts_safe = lax.slice(aux, (0, 0), (_NSEG, 1))
    rep_safe = lax.slice(aux, (0, 1), (_NSEG, 2))
    has_rep = lax.slice(aux, (0, 2), (_NSEG, 3))
    gfv = gfv_ref[0]
    attn_contrib = jnp.sum(lax.slice(gfv, (0, 4), (1, 5)))

    r256 = lax.broadcasted_iota(jnp.int32, (_NSEG, _NQ * _LSEG), 0)
    cq = lax.broadcasted_iota(jnp.int32, (_NSEG, _NQ * _LSEG), 1)
    l_of = cq % _LSEG
    gf_of = jnp.full((_NSEG, _NQ * _LSEG), _BIG, jnp.int32)
    for q in range(_NQ):
        gq = jnp.sum(lax.slice(gfv, (0, q), (1, q + 1))).astype(jnp.int32)
        gf_of = jnp.where(cq // _LSEG == q, gq, gf_of)
    mq = jnp.where(r256 == gf_of + l_of, 1.0, 0.0)

    cp.wait()
    acc_p = jnp.dot(
        mq, jnp.squeeze(accv_ref[0], axis=1), preferred_element_type=jnp.float32
    )
    acc_r = jnp.dot(
        mq, jnp.squeeze(accv_ref[1], axis=1), preferred_element_type=jnp.float32
    )

    diff = acc_p / counts_safe - acc_r / rep_safe
    mse = jnp.sum(diff * diff, axis=1, keepdims=True) / _D
    feat_contrib = jnp.sum(has_rep * mse)
    cnt_contrib = jnp.sum(has_rep)

    sacc_ref[0] += feat_contrib
    sacc_ref[1] += cnt_contrib
    sacc_ref[2] += attn_contrib

    @pl.when(bidx == nb - 1)
    def _fin():
        fls = sacc_ref[0]
        fc = sacc_ref[1]
        feat_loss = jnp.where(fc > 0.0, fls / jnp.maximum(fc, 1.0), fls)
        out_ref[0, 0] = _W_FEAT * feat_loss + _W_ATTN * sacc_ref[2] / nb


def kernel(attn, feat):
    B = attn.shape[0]
    attn3 = attn.reshape(B, 1, _T)

    idx_w, aux, gfv = pl.pallas_call(
        _prep_kernel,
        in_specs=[pl.BlockSpec((B, 1, _T), lambda: (0, 0, 0))],
        out_specs=[
            pl.BlockSpec((_NQ, B, 2 * _QT), lambda: (0, 0, 0)),
            pl.BlockSpec((B, _NSEG, 128), lambda: (0, 0, 0)),
            pl.BlockSpec((B, 1, 128), lambda: (0, 0, 0)),
        ],
        out_shape=[
            jax.ShapeDtypeStruct((_NQ, B, 2 * _QT), jnp.int32),
            jax.ShapeDtypeStruct((B, _NSEG, 128), jnp.float32),
            jax.ShapeDtypeStruct((B, 1, 128), jnp.float32),
        ],
    )(attn3)

    feat_flat = feat.reshape(B, _T * _D)
    acc5 = _sc_scatter(feat_flat, idx_w)

    out = pl.pallas_call(
        _finalize_kernel,
        grid=(B,),
        in_specs=[
            pl.BlockSpec((1, _NSEG, 128), lambda b: (b, 0, 0)),
            pl.BlockSpec((1, 1, 128), lambda b: (b, 0, 0)),
            pl.BlockSpec(memory_space=pl.ANY),
        ],
        out_specs=pl.BlockSpec(memory_space=pltpu.SMEM),
        out_shape=jax.ShapeDtypeStruct((1, 1), jnp.float32),
        scratch_shapes=[
            pltpu.SMEM((3,), jnp.float32),
            pltpu.VMEM((2, _NQ * _LSEG, 1, _D), jnp.float32),
            pltpu.SemaphoreType.DMA,
        ],
    )(aux, gfv, acc5)
    return out[0, 0]


# speedup vs baseline: 1.8530x; 1.4311x over previous
import jax
import jax.numpy as jnp
from jax import lax
from jax.experimental import pallas as pl
from jax.experimental.pallas import tpu as pltpu
from jax.experimental.pallas import tpu_sc as plsc

_P_THR = 0.55
_C_THR = 0.7
_W_FEAT = 1.0
_W_ATTN = 1.0

_T = 512
_D = 512
_NSEG = 256
_NQ = 4
_QT = _T // _NQ
_LSEG = 64
_TRASH = _LSEG
_BIG = 1 << 20
_CH = 32


def _prep_kernel(attn_ref, idx_ref, aux_ref, gfv_ref):
    B = attn_ref.shape[0]
    a = attn_ref[:, 0, :]
    pred = a > _P_THR
    pred_f = jnp.where(pred, 1.0, 0.0)
    r = lax.broadcasted_iota(jnp.int32, (_T, _T), 0)
    c = lax.broadcasted_iota(jnp.int32, (_T, _T), 1)
    shift = jnp.where(r + 1 == c, 1.0, 0.0)
    triu = jnp.where(r <= c, 1.0, 0.0)
    prev_f = jnp.dot(pred_f, shift, preferred_element_type=jnp.float32)
    start_f = pred_f * (1.0 - prev_f)
    cum = jnp.dot(start_f, triu, preferred_element_type=jnp.float32)
    seg = cum.astype(jnp.int32) - 1
    col = lax.broadcasted_iota(jnp.int32, (B, _T), 1)
    rep = a > _C_THR

    segm = jnp.where(pred, seg, _BIG)
    gfs = []
    for q in range(_NQ):
        mask_q = (col >= q * _QT) & (col < (q + 1) * _QT)
        gfs.append(jnp.min(jnp.where(mask_q, segm, _BIG), axis=1, keepdims=True))

    q_of_col = col // _QT
    gf_vec = jnp.full((B, _T), _BIG, jnp.int32)
    for q in range(_NQ):
        gf_vec = jnp.where(q_of_col == q, gfs[q], gf_vec)
    l = seg - gf_vec
    idx_p = jnp.where(pred, l, _TRASH)
    idx_r = jnp.where(rep, l, _TRASH)
    for q in range(_NQ):
        p_sl = lax.slice(idx_p, (0, q * _QT), (B, (q + 1) * _QT))
        r_sl = lax.slice(idx_r, (0, q * _QT), (B, (q + 1) * _QT))
        idx_ref[q] = jnp.concatenate([p_sl, r_sl], axis=1)

    row_ids = lax.broadcasted_iota(jnp.int32, (_NSEG, _T), 0)
    lane = lax.broadcasted_iota(jnp.int32, (_NSEG, 128), 1)
    lane1 = lax.broadcasted_iota(jnp.int32, (1, 128), 1)
    rep_f = jnp.where(rep, 1.0, 0.0)
    for b in range(B):
        seg_b = lax.slice(seg, (b, 0), (b + 1, _T))
        pred_b = lax.slice(pred_f, (b, 0), (b + 1, _T))
        rep_b = lax.slice(rep_f, (b, 0), (b + 1, _T))
        a_b = lax.slice(a, (b, 0), (b + 1, _T))
        m = jnp.where(row_ids == seg_b, 1.0, 0.0) * pred_b
        m_rep = m * rep_b
        counts = jnp.sum(m, axis=1, keepdims=True)
        rep_counts = jnp.sum(m_rep, axis=1, keepdims=True)
        sum_a = jnp.sum(m * a_b, axis=1, keepdims=True)
        sum_a2 = jnp.sum(m * (a_b * a_b), axis=1, keepdims=True)
        valid = counts > 0.0
        counts_safe = jnp.where(valid, counts, 1.0)
        mean_a = sum_a / counts_safe
        var = sum_a2 / counts_safe - mean_a * mean_a
        nprop = jnp.sum(jnp.where(valid, 1.0, 0.0))
        video_loss = jnp.sum(jnp.where(valid, var, 0.0))
        attn_contrib = jnp.where(
            nprop > 0.0, video_loss / jnp.maximum(nprop, 1.0), 0.0
        )
        has_rep = jnp.where(valid & (rep_counts > 0.0), 1.0, 0.0)
        rep_safe = jnp.where(has_rep > 0.0, rep_counts, 1.0)
        aux = (
            jnp.where(lane == 0, counts_safe, 0.0)
            + jnp.where(lane == 1, rep_safe, 0.0)
            + jnp.where(lane == 2, has_rep, 0.0)
        )
        aux_ref[b] = aux
        gfv = jnp.where(lane1 == 4, attn_contrib, 0.0)
        for q in range(_NQ):
            gq = jnp.sum(lax.slice(gfs[q], (b, 0), (b + 1, 1)))
            gfv = jnp.where(lane1 == q, gq.astype(jnp.float32), gfv)
        gfv_ref[b] = gfv


def _sc_body(feat_hbm, idx_hbm, out_hbm, acc, idxb, stage):
    c = lax.axis_index("c")
    s = lax.axis_index("s")
    w = c * 16 + s
    b = w // _NQ
    q = w % _NQ

    pltpu.sync_copy(idx_hbm.at[q, b], idxb)

    zero16 = jnp.zeros((16,), jnp.float32)
    nzrows = 2 * _LSEG

    @plsc.parallel_loop(0, nzrows, 1, unroll=2)
    def _zrow(i):
        m = i // _LSEG
        row = i % _LSEG
        for j in range(_D // 16):
            acc[m, row, 0, pl.ds(j * 16, 16)] = zero16

    def chunk_body(k, carry):
        pltpu.sync_copy(
            feat_hbm.at[b, pl.ds(q * _QT * _D + k * (_CH * _D), _CH * _D)], stage
        )
        for g in range(_CH // 16):
            ipv = idxb[pl.ds(k * _CH + g * 16, 16)]
            irv = idxb[pl.ds(_QT + k * _CH + g * 16, 16)]
            ips = [ipv[r2] for r2 in range(16)]
            irs = [irv[r2] for r2 in range(16)]

            @plsc.parallel_loop(0, _D // 16, 1, unroll=4)
            def _jloop(j, g=g, ips=ips, irs=irs):
                off = j * 16
                for r2 in range(16):
                    r = g * 16 + r2
                    v = stage[pl.ds(r * _D + off, 16)]
                    plsc.addupdate(acc.at[0, ips[r2], 0, pl.ds(off, 16)], v)
                    plsc.addupdate(acc.at[1, irs[r2], 0, pl.ds(off, 16)], v)

        return carry

    lax.fori_loop(0, _QT // _CH, chunk_body, 0)

    pltpu.sync_copy(acc.at[0, pl.ds(0, _LSEG)], out_hbm.at[b, 0, pl.ds(q * _LSEG, _LSEG)])
    pltpu.sync_copy(acc.at[1, pl.ds(0, _LSEG)], out_hbm.at[b, 1, pl.ds(q * _LSEG, _LSEG)])


def _sc_scatter(feat_flat, idx_w):
    run = pl.kernel(
        _sc_body,
        out_type=jax.ShapeDtypeStruct((8, 2, _NQ * _LSEG, 1, _D), jnp.float32),
        mesh=plsc.VectorSubcoreMesh(
            core_axis_name="c", subcore_axis_name="s", num_cores=2, num_subcores=16
        ),
        scratch_types=[
            pltpu.VMEM((2, _LSEG + 1, 1, _D), jnp.float32),
            pltpu.VMEM((2 * _QT,), jnp.int32),
            pltpu.VMEM((_CH * _D,), jnp.float32),
        ],
    )
    return run(feat_flat, idx_w)


def _finalize_kernel(aux_ref, gfv_ref, acc_hbm, out_ref, sacc_ref, accv_ref, sem):
    bidx = pl.program_id(0)
    nb = pl.num_programs(0)

    cp = pltpu.make_async_copy(acc_hbm.at[bidx], accv_ref, sem)
    cp.start()

    @pl.when(bidx == 0)
    def _init():
        sacc_ref[0] = 0.0
        sacc_ref[1] = 0.0
        sacc_ref[2] = 0.0

    aux = aux_ref[0]
    counts_safe = lax.slice(aux, (0, 0), (_NSEG, 1))
    rep_safe = lax.slice(aux, (0, 1), (_NSEG, 2))
    has_rep = lax.slice(aux, (0, 2), (_NSEG, 3))
    gfv = gfv_ref[0]
    attn_contrib = jnp.sum(lax.slice(gfv, (0, 4), (1, 5)))

    r256 = lax.broadcasted_iota(jnp.int32, (_NSEG, _NQ * _LSEG), 0)
    cq = lax.broadcasted_iota(jnp.int32, (_NSEG, _NQ * _LSEG), 1)
    l_of = cq % _LSEG
    gf_of = jnp.full((_NSEG, _NQ * _LSEG), _BIG, jnp.int32)
    for q in range(_NQ):
        gq = jnp.sum(lax.slice(gfv, (0, q), (1, q + 1))).astype(jnp.int32)
        gf_of = jnp.where(cq // _LSEG == q, gq, gf_of)
    mq = jnp.where(r256 == gf_of + l_of, 1.0, 0.0)

    cp.wait()
    acc_p = jnp.dot(
        mq, jnp.squeeze(accv_ref[0], axis=1), preferred_element_type=jnp.float32
    )
    acc_r = jnp.dot(
        mq, jnp.squeeze(accv_ref[1], axis=1), preferred_element_type=jnp.float32
    )

    diff = acc_p / counts_safe - acc_r / rep_safe
    mse = jnp.sum(diff * diff, axis=1, keepdims=True) / _D
    feat_contrib = jnp.sum(has_rep * mse)
    cnt_contrib = jnp.sum(has_rep)

    sacc_ref[0] += feat_contrib
    sacc_ref[1] += cnt_contrib
    sacc_ref[2] += attn_contrib

    @pl.when(bidx == nb - 1)
    def _fin():
        fls = sacc_ref[0]
        fc = sacc_ref[1]
        feat_loss = jnp.where(fc > 0.0, fls / jnp.maximum(fc, 1.0), fls)
        out_ref[0, 0] = _W_FEAT * feat_loss + _W_ATTN * sacc_ref[2] / nb


def kernel(attn, feat):
    B = attn.shape[0]
    attn3 = attn.reshape(B, 1, _T)

    idx_w, aux, gfv = pl.pallas_call(
        _prep_kernel,
        in_specs=[pl.BlockSpec((B, 1, _T), lambda: (0, 0, 0))],
        out_specs=[
            pl.BlockSpec((_NQ, B, 2 * _QT), lambda: (0, 0, 0)),
            pl.BlockSpec((B, _NSEG, 128), lambda: (0, 0, 0)),
            pl.BlockSpec((B, 1, 128), lambda: (0, 0, 0)),
        ],
        out_shape=[
            jax.ShapeDtypeStruct((_NQ, B, 2 * _QT), jnp.int32),
            jax.ShapeDtypeStruct((B, _NSEG, 128), jnp.float32),
            jax.ShapeDtypeStruct((B, 1, 128), jnp.float32),
        ],
    )(attn3)

    feat_flat = feat.reshape(B, _T * _D)
    acc5 = _sc_scatter(feat_flat, idx_w)

    out = pl.pallas_call(
        _finalize_kernel,
        grid=(B,),
        in_specs=[
            pl.BlockSpec((1, _NSEG, 128), lambda b: (b, 0, 0)),
            pl.BlockSpec((1, 1, 128), lambda b: (b, 0, 0)),
            pl.BlockSpec(memory_space=pl.ANY),
        ],
        out_specs=pl.BlockSpec(memory_space=pltpu.SMEM),
        out_shape=jax.ShapeDtypeStruct((1, 1), jnp.float32),
        scratch_shapes=[
            pltpu.SMEM((3,), jnp.float32),
            pltpu.VMEM((2, _NQ * _LSEG, 1, _D), jnp.float32),
            pltpu.SemaphoreType.DMA,
        ],
    )(aux, gfv, acc5)
    return out[0, 0]


# speedup vs baseline: 2.0587x; 1.1110x over previous
import jax
import jax.numpy as jnp
from jax import lax
from jax.experimental import pallas as pl
from jax.experimental.pallas import tpu as pltpu
from jax.experimental.pallas import tpu_sc as plsc

_P_THR = 0.55
_C_THR = 0.7
_W_FEAT = 1.0
_W_ATTN = 1.0

_T = 512
_D = 512
_NSEG = 256
_NQ = 4
_QT = _T // _NQ
_LSEG = 64
_TRASH = _LSEG
_BIG = 1 << 20
_CH = 32


def _prep_kernel(attn_ref, idx_ref, aux_ref, gfv_ref):
    B = attn_ref.shape[0]
    a = attn_ref[:, 0, :]
    pred = a > _P_THR
    pred_f = jnp.where(pred, 1.0, 0.0)
    r = lax.broadcasted_iota(jnp.int32, (_T, _T), 0)
    c = lax.broadcasted_iota(jnp.int32, (_T, _T), 1)
    shift = jnp.where(r + 1 == c, 1.0, 0.0)
    triu = jnp.where(r <= c, 1.0, 0.0)
    prev_f = jnp.dot(pred_f, shift, preferred_element_type=jnp.float32)
    start_f = pred_f * (1.0 - prev_f)
    cum = jnp.dot(start_f, triu, preferred_element_type=jnp.float32)
    seg = cum.astype(jnp.int32) - 1
    col = lax.broadcasted_iota(jnp.int32, (B, _T), 1)
    rep = a > _C_THR

    segm = jnp.where(pred, seg, _BIG)
    gfs = []
    for q in range(_NQ):
        mask_q = (col >= q * _QT) & (col < (q + 1) * _QT)
        gfs.append(jnp.min(jnp.where(mask_q, segm, _BIG), axis=1, keepdims=True))

    q_of_col = col // _QT
    gf_vec = jnp.full((B, _T), _BIG, jnp.int32)
    for q in range(_NQ):
        gf_vec = jnp.where(q_of_col == q, gfs[q], gf_vec)
    l = seg - gf_vec
    idx_p = jnp.where(pred, l, _TRASH)
    idx_r = jnp.where(rep, l, _TRASH)
    for q in range(_NQ):
        p_sl = lax.slice(idx_p, (0, q * _QT), (B, (q + 1) * _QT))
        r_sl = lax.slice(idx_r, (0, q * _QT), (B, (q + 1) * _QT))
        idx_ref[q] = jnp.concatenate([p_sl, r_sl], axis=1)

    row_ids = lax.broadcasted_iota(jnp.int32, (_NSEG, _T), 0)
    lane = lax.broadcasted_iota(jnp.int32, (_NSEG, 128), 1)
    lane1 = lax.broadcasted_iota(jnp.int32, (1, 128), 1)
    rep_f = jnp.where(rep, 1.0, 0.0)
    for b in range(B):
        seg_b = lax.slice(seg, (b, 0), (b + 1, _T))
        pred_b = lax.slice(pred_f, (b, 0), (b + 1, _T))
        rep_b = lax.slice(rep_f, (b, 0), (b + 1, _T))
        a_b = lax.slice(a, (b, 0), (b + 1, _T))
        m = jnp.where(row_ids == seg_b, 1.0, 0.0) * pred_b
        m_rep = m * rep_b
        counts = jnp.sum(m, axis=1, keepdims=True)
        rep_counts = jnp.sum(m_rep, axis=1, keepdims=True)
        sum_a = jnp.sum(m * a_b, axis=1, keepdims=True)
        sum_a2 = jnp.sum(m * (a_b * a_b), axis=1, keepdims=True)
        valid = counts > 0.0
        counts_safe = jnp.where(valid, counts, 1.0)
        mean_a = sum_a / counts_safe
        var = sum_a2 / counts_safe - mean_a * mean_a
        nprop = jnp.sum(jnp.where(valid, 1.0, 0.0))
        video_loss = jnp.sum(jnp.where(valid, var, 0.0))
        attn_contrib = jnp.where(
            nprop > 0.0, video_loss / jnp.maximum(nprop, 1.0), 0.0
        )
        has_rep = jnp.where(valid & (rep_counts > 0.0), 1.0, 0.0)
        rep_safe = jnp.where(has_rep > 0.0, rep_counts, 1.0)
        aux = (
            jnp.where(lane == 0, counts_safe, 0.0)
            + jnp.where(lane == 1, rep_safe, 0.0)
            + jnp.where(lane == 2, has_rep, 0.0)
        )
        aux_ref[b] = aux
        gfv = jnp.where(lane1 == 4, attn_contrib, 0.0)
        for q in range(_NQ):
            gq = jnp.sum(lax.slice(gfs[q], (b, 0), (b + 1, 1)))
            gfv = jnp.where(lane1 == q, gq.astype(jnp.float32), gfv)
        gfv_ref[b] = gfv


def _sc_body(feat_hbm, idx_hbm, out_hbm, acc, idxb, stage0, stage1, sem0, sem1):
    c = lax.axis_index("c")
    s = lax.axis_index("s")
    w = c * 16 + s
    b = w // _NQ
    q = w % _NQ

    pltpu.sync_copy(idx_hbm.at[q, b], idxb)

    zero16 = jnp.zeros((16,), jnp.float32)
    nzrows = 2 * _LSEG

    @plsc.parallel_loop(0, nzrows, 1, unroll=2)
    def _zrow(i):
        m = i // _LSEG
        row = i % _LSEG
        for j in range(_D // 16):
            acc[m, row, 0, pl.ds(j * 16, 16)] = zero16

    def start_chunk(k, buf, sem):
        return pltpu.async_copy(
            feat_hbm.at[b, pl.ds(q * _QT * _D + k * (_CH * _D), _CH * _D)], buf, sem
        )

    def process_chunk(k, stage):
        for g in range(_CH // 16):
            ipv = idxb[pl.ds(k * _CH + g * 16, 16)]
            irv = idxb[pl.ds(_QT + k * _CH + g * 16, 16)]
            ips = [ipv[r2] for r2 in range(16)]
            irs = [irv[r2] for r2 in range(16)]

            @plsc.parallel_loop(0, _D // 16, 1, unroll=4)
            def _jloop(j, g=g, ips=ips, irs=irs):
                off = j * 16
                for r2 in range(16):
                    r = g * 16 + r2
                    v = stage[pl.ds(r * _D + off, 16)]
                    plsc.addupdate(acc.at[0, ips[r2], 0, pl.ds(off, 16)], v)
                    plsc.addupdate(acc.at[1, irs[r2], 0, pl.ds(off, 16)], v)

    start_chunk(0, stage0, sem0)

    def pair_body(p, carry):
        start_chunk(2 * p + 1, stage1, sem1)
        pltpu.make_async_copy(
            feat_hbm.at[b, pl.ds(0, _CH * _D)], stage0, sem0
        ).wait()
        process_chunk(2 * p, stage0)

        @pl.when(p < _QT // _CH // 2 - 1)
        def _next():
            start_chunk(2 * p + 2, stage0, sem0)

        pltpu.make_async_copy(
            feat_hbm.at[b, pl.ds(0, _CH * _D)], stage1, sem1
        ).wait()
        process_chunk(2 * p + 1, stage1)
        return carry

    lax.fori_loop(0, _QT // _CH // 2, pair_body, 0)

    pltpu.sync_copy(acc.at[0, pl.ds(0, _LSEG)], out_hbm.at[b, 0, pl.ds(q * _LSEG, _LSEG)])
    pltpu.sync_copy(acc.at[1, pl.ds(0, _LSEG)], out_hbm.at[b, 1, pl.ds(q * _LSEG, _LSEG)])


def _sc_scatter(feat_flat, idx_w):
    run = pl.kernel(
        _sc_body,
        out_type=jax.ShapeDtypeStruct((8, 2, _NQ * _LSEG, 1, _D), jnp.float32),
        mesh=plsc.VectorSubcoreMesh(
            core_axis_name="c", subcore_axis_name="s", num_cores=2, num_subcores=16
        ),
        scratch_types=[
            pltpu.VMEM((2, _LSEG + 1, 1, _D), jnp.float32),
            pltpu.VMEM((2 * _QT,), jnp.int32),
            pltpu.VMEM((_CH * _D,), jnp.float32),
            pltpu.VMEM((_CH * _D,), jnp.float32),
            pltpu.SemaphoreType.DMA,
            pltpu.SemaphoreType.DMA,
        ],
    )
    return run(feat_flat, idx_w)


def _finalize_kernel(aux_ref, gfv_ref, acc_hbm, out_ref, sacc_ref, accv_ref, sem0, sem1):
    bidx = pl.program_id(0)
    nb = pl.num_programs(0)
    par = lax.rem(bidx, 2)

    @pl.when(bidx == 0)
    def _first():
        pltpu.make_async_copy(acc_hbm.at[0], accv_ref.at[0], sem0).start()
        sacc_ref[0] = 0.0
        sacc_ref[1] = 0.0
        sacc_ref[2] = 0.0

    @pl.when((bidx < nb - 1) & (par == 0))
    def _pf0():
        pltpu.make_async_copy(acc_hbm.at[bidx + 1], accv_ref.at[1], sem1).start()

    @pl.when((bidx < nb - 1) & (par == 1))
    def _pf1():
        pltpu.make_async_copy(acc_hbm.at[bidx + 1], accv_ref.at[0], sem0).start()

    aux = aux_ref[0]
    counts_safe = lax.slice(aux, (0, 0), (_NSEG, 1))
    rep_safe = lax.slice(aux, (0, 1), (_NSEG, 2))
    has_rep = lax.slice(aux, (0, 2), (_NSEG, 3))
    gfv = gfv_ref[0]
    attn_contrib = jnp.sum(lax.slice(gfv, (0, 4), (1, 5)))

    r256 = lax.broadcasted_iota(jnp.int32, (_NSEG, _NQ * _LSEG), 0)
    cq = lax.broadcasted_iota(jnp.int32, (_NSEG, _NQ * _LSEG), 1)
    l_of = cq % _LSEG
    gf_of = jnp.full((_NSEG, _NQ * _LSEG), _BIG, jnp.int32)
    for q in range(_NQ):
        gq = jnp.sum(lax.slice(gfv, (0, q), (1, q + 1))).astype(jnp.int32)
        gf_of = jnp.where(cq // _LSEG == q, gq, gf_of)
    mq = jnp.where(r256 == gf_of + l_of, 1.0, 0.0)

    @pl.when(par == 0)
    def _w0():
        pltpu.make_async_copy(acc_hbm.at[0], accv_ref.at[0], sem0).wait()

    @pl.when(par == 1)
    def _w1():
        pltpu.make_async_copy(acc_hbm.at[0], accv_ref.at[1], sem1).wait()

    acc_p = jnp.dot(
        mq, jnp.squeeze(accv_ref[par, 0], axis=1), preferred_element_type=jnp.float32
    )
    acc_r = jnp.dot(
        mq, jnp.squeeze(accv_ref[par, 1], axis=1), preferred_element_type=jnp.float32
    )

    diff = acc_p / counts_safe - acc_r / rep_safe
    mse = jnp.sum(diff * diff, axis=1, keepdims=True) / _D
    feat_contrib = jnp.sum(has_rep * mse)
    cnt_contrib = jnp.sum(has_rep)

    sacc_ref[0] += feat_contrib
    sacc_ref[1] += cnt_contrib
    sacc_ref[2] += attn_contrib

    @pl.when(bidx == nb - 1)
    def _fin():
        fls = sacc_ref[0]
        fc = sacc_ref[1]
        feat_loss = jnp.where(fc > 0.0, fls / jnp.maximum(fc, 1.0), fls)
        out_ref[0, 0] = _W_FEAT * feat_loss + _W_ATTN * sacc_ref[2] / nb


def kernel(attn, feat):
    B = attn.shape[0]
    attn3 = attn.reshape(B, 1, _T)

    idx_w, aux, gfv = pl.pallas_call(
        _prep_kernel,
        in_specs=[pl.BlockSpec((B, 1, _T), lambda: (0, 0, 0))],
        out_specs=[
            pl.BlockSpec((_NQ, B, 2 * _QT), lambda: (0, 0, 0)),
            pl.BlockSpec((B, _NSEG, 128), lambda: (0, 0, 0)),
            pl.BlockSpec((B, 1, 128), lambda: (0, 0, 0)),
        ],
        out_shape=[
            jax.ShapeDtypeStruct((_NQ, B, 2 * _QT), jnp.int32),
            jax.ShapeDtypeStruct((B, _NSEG, 128), jnp.float32),
            jax.ShapeDtypeStruct((B, 1, 128), jnp.float32),
        ],
    )(attn3)

    feat_flat = feat.reshape(B, _T * _D)
    acc5 = _sc_scatter(feat_flat, idx_w)

    out = pl.pallas_call(
        _finalize_kernel,
        grid=(B,),
        in_specs=[
            pl.BlockSpec((1, _NSEG, 128), lambda b: (b, 0, 0)),
            pl.BlockSpec((1, 1, 128), lambda b: (b, 0, 0)),
            pl.BlockSpec(memory_space=pl.ANY),
        ],
        out_specs=pl.BlockSpec(memory_space=pltpu.SMEM),
        out_shape=jax.ShapeDtypeStruct((1, 1), jnp.float32),
        scratch_shapes=[
            pltpu.SMEM((3,), jnp.float32),
            pltpu.VMEM((2, 2, _NQ * _LSEG, 1, _D), jnp.float32),
            pltpu.SemaphoreType.DMA,
            pltpu.SemaphoreType.DMA,
        ],
    )(aux, gfv, acc5)
    return out[0, 0]


# speedup vs baseline: 2.0844x; 1.0125x over previous
import jax
import jax.numpy as jnp
from jax import lax
from jax.experimental import pallas as pl
from jax.experimental.pallas import tpu as pltpu
from jax.experimental.pallas import tpu_sc as plsc

_P_THR = 0.55
_C_THR = 0.7
_W_FEAT = 1.0
_W_ATTN = 1.0

_T = 512
_D = 512
_NSEG = 256
_NQ = 4
_QT = _T // _NQ
_LSEG = 64
_TRASH = _LSEG
_BIG = 1 << 20
_CH = 32


def _prep_kernel(attn_ref, idx_ref, aux_ref, gfv_ref):
    B = attn_ref.shape[0]
    a = attn_ref[:, 0, :]
    pred = a > _P_THR
    pred_f = jnp.where(pred, 1.0, 0.0)
    r = lax.broadcasted_iota(jnp.int32, (_T, _T), 0)
    c = lax.broadcasted_iota(jnp.int32, (_T, _T), 1)
    shift = jnp.where(r + 1 == c, 1.0, 0.0)
    triu = jnp.where(r <= c, 1.0, 0.0)
    prev_f = jnp.dot(pred_f, shift, preferred_element_type=jnp.float32)
    start_f = pred_f * (1.0 - prev_f)
    cum = jnp.dot(start_f, triu, preferred_element_type=jnp.float32)
    seg = cum.astype(jnp.int32) - 1
    col = lax.broadcasted_iota(jnp.int32, (B, _T), 1)
    rep = a > _C_THR

    segm = jnp.where(pred, seg, _BIG)
    gfs = []
    for q in range(_NQ):
        mask_q = (col >= q * _QT) & (col < (q + 1) * _QT)
        gfs.append(jnp.min(jnp.where(mask_q, segm, _BIG), axis=1, keepdims=True))

    q_of_col = col // _QT
    gf_vec = jnp.full((B, _T), _BIG, jnp.int32)
    for q in range(_NQ):
        gf_vec = jnp.where(q_of_col == q, gfs[q], gf_vec)
    l = seg - gf_vec
    idx_p = jnp.where(pred, l, _TRASH)
    idx_r = jnp.where(rep, l, _TRASH)
    for q in range(_NQ):
        p_sl = lax.slice(idx_p, (0, q * _QT), (B, (q + 1) * _QT))
        r_sl = lax.slice(idx_r, (0, q * _QT), (B, (q + 1) * _QT))
        idx_ref[q] = jnp.concatenate([p_sl, r_sl], axis=1)

    row_ids = lax.broadcasted_iota(jnp.int32, (_NSEG, _T), 0)
    lane = lax.broadcasted_iota(jnp.int32, (_NSEG, 128), 1)
    lane1 = lax.broadcasted_iota(jnp.int32, (1, 128), 1)
    rep_f = jnp.where(rep, 1.0, 0.0)
    for b in range(B):
        seg_b = lax.slice(seg, (b, 0), (b + 1, _T))
        pred_b = lax.slice(pred_f, (b, 0), (b + 1, _T))
        rep_b = lax.slice(rep_f, (b, 0), (b + 1, _T))
        a_b = lax.slice(a, (b, 0), (b + 1, _T))
        m = jnp.where(row_ids == seg_b, 1.0, 0.0) * pred_b
        m_rep = m * rep_b
        counts = jnp.sum(m, axis=1, keepdims=True)
        rep_counts = jnp.sum(m_rep, axis=1, keepdims=True)
        sum_a = jnp.sum(m * a_b, axis=1, keepdims=True)
        sum_a2 = jnp.sum(m * (a_b * a_b), axis=1, keepdims=True)
        valid = counts > 0.0
        counts_safe = jnp.where(valid, counts, 1.0)
        mean_a = sum_a / counts_safe
        var = sum_a2 / counts_safe - mean_a * mean_a
        nprop = jnp.sum(jnp.where(valid, 1.0, 0.0))
        video_loss = jnp.sum(jnp.where(valid, var, 0.0))
        attn_contrib = jnp.where(
            nprop > 0.0, video_loss / jnp.maximum(nprop, 1.0), 0.0
        )
        has_rep = jnp.where(valid & (rep_counts > 0.0), 1.0, 0.0)
        rep_safe = jnp.where(has_rep > 0.0, rep_counts, 1.0)
        aux = (
            jnp.where(lane == 0, counts_safe, 0.0)
            + jnp.where(lane == 1, rep_safe, 0.0)
            + jnp.where(lane == 2, has_rep, 0.0)
        )
        aux_ref[b] = aux
        gfv = jnp.where(lane1 == 4, attn_contrib, 0.0)
        for q in range(_NQ):
            gq = jnp.sum(lax.slice(gfs[q], (b, 0), (b + 1, 1)))
            gfv = jnp.where(lane1 == q, gq.astype(jnp.float32), gfv)
        gfv_ref[b] = gfv


def _sc_body(feat_hbm, idx_hbm, out_hbm, acc, idxb, stage0, stage1, sem0, sem1):
    c = lax.axis_index("c")
    s = lax.axis_index("s")
    w = c * 16 + s
    b = w // _NQ
    q = w % _NQ

    pltpu.sync_copy(idx_hbm.at[q, b], idxb)

    zero16 = jnp.zeros((16,), jnp.float32)
    nzrows = 2 * _LSEG

    @plsc.parallel_loop(0, nzrows, 1, unroll=2)
    def _zrow(i):
        m = i // _LSEG
        row = i % _LSEG
        for j in range(_D // 16):
            acc[m, row, 0, pl.ds(j * 16, 16)] = zero16

    def start_chunk(k, buf, sem):
        return pltpu.async_copy(
            feat_hbm.at[b, pl.ds(q * _QT * _D + k * (_CH * _D), _CH * _D)], buf, sem
        )

    def process_chunk(k, stage):
        for g in range(_CH // 16):
            ipv = idxb[pl.ds(k * _CH + g * 16, 16)]
            irv = idxb[pl.ds(_QT + k * _CH + g * 16, 16)]
            ips = [ipv[r2] for r2 in range(16)]
            irs = [irv[r2] for r2 in range(16)]

            @plsc.parallel_loop(0, _D // 16, 1, unroll=8)
            def _jloop(j, g=g, ips=ips, irs=irs):
                off = j * 16
                for r2 in range(16):
                    r = g * 16 + r2
                    v = stage[pl.ds(r * _D + off, 16)]
                    plsc.addupdate(acc.at[0, ips[r2], 0, pl.ds(off, 16)], v)
                    plsc.addupdate(acc.at[1, irs[r2], 0, pl.ds(off, 16)], v)

    start_chunk(0, stage0, sem0)

    def pair_body(p, carry):
        start_chunk(2 * p + 1, stage1, sem1)
        pltpu.make_async_copy(
            feat_hbm.at[b, pl.ds(0, _CH * _D)], stage0, sem0
        ).wait()
        process_chunk(2 * p, stage0)

        @pl.when(p < _QT // _CH // 2 - 1)
        def _next():
            start_chunk(2 * p + 2, stage0, sem0)

        pltpu.make_async_copy(
            feat_hbm.at[b, pl.ds(0, _CH * _D)], stage1, sem1
        ).wait()
        process_chunk(2 * p + 1, stage1)
        return carry

    lax.fori_loop(0, _QT // _CH // 2, pair_body, 0)

    pltpu.sync_copy(acc.at[0, pl.ds(0, _LSEG)], out_hbm.at[b, 0, pl.ds(q * _LSEG, _LSEG)])
    pltpu.sync_copy(acc.at[1, pl.ds(0, _LSEG)], out_hbm.at[b, 1, pl.ds(q * _LSEG, _LSEG)])


def _sc_scatter(feat_flat, idx_w):
    run = pl.kernel(
        _sc_body,
        out_type=jax.ShapeDtypeStruct((8, 2, _NQ * _LSEG, 1, _D), jnp.float32),
        mesh=plsc.VectorSubcoreMesh(
            core_axis_name="c", subcore_axis_name="s", num_cores=2, num_subcores=16
        ),
        scratch_types=[
            pltpu.VMEM((2, _LSEG + 1, 1, _D), jnp.float32),
            pltpu.VMEM((2 * _QT,), jnp.int32),
            pltpu.VMEM((_CH * _D,), jnp.float32),
            pltpu.VMEM((_CH * _D,), jnp.float32),
            pltpu.SemaphoreType.DMA,
            pltpu.SemaphoreType.DMA,
        ],
    )
    return run(feat_flat, idx_w)


def _finalize_kernel(aux_ref, gfv_ref, acc_hbm, out_ref, sacc_ref, accv_ref, sem0, sem1):
    bidx = pl.program_id(0)
    nb = pl.num_programs(0)
    par = lax.rem(bidx, 2)

    @pl.when(bidx == 0)
    def _first():
        pltpu.make_async_copy(acc_hbm.at[0], accv_ref.at[0], sem0).start()
        sacc_ref[0] = 0.0
        sacc_ref[1] = 0.0
        sacc_ref[2] = 0.0

    @pl.when((bidx < nb - 1) & (par == 0))
    def _pf0():
        pltpu.make_async_copy(acc_hbm.at[bidx + 1], accv_ref.at[1], sem1).start()

    @pl.when((bidx < nb - 1) & (par == 1))
    def _pf1():
        pltpu.make_async_copy(acc_hbm.at[bidx + 1], accv_ref.at[0], sem0).start()

    aux = aux_ref[0]
    counts_safe = lax.slice(aux, (0, 0), (_NSEG, 1))
    rep_safe = lax.slice(aux, (0, 1), (_NSEG, 2))
    has_rep = lax.slice(aux, (0, 2), (_NSEG, 3))
    gfv = gfv_ref[0]
    attn_contrib = jnp.sum(lax.slice(gfv, (0, 4), (1, 5)))

    r256 = lax.broadcasted_iota(jnp.int32, (_NSEG, _NQ * _LSEG), 0)
    cq = lax.broadcasted_iota(jnp.int32, (_NSEG, _NQ * _LSEG), 1)
    l_of = cq % _LSEG
    gf_of = jnp.full((_NSEG, _NQ * _LSEG), _BIG, jnp.int32)
    for q in range(_NQ):
        gq = jnp.sum(lax.slice(gfv, (0, q), (1, q + 1))).astype(jnp.int32)
        gf_of = jnp.where(cq // _LSEG == q, gq, gf_of)
    mq = jnp.where(r256 == gf_of + l_of, 1.0, 0.0)

    @pl.when(par == 0)
    def _w0():
        pltpu.make_async_copy(acc_hbm.at[0], accv_ref.at[0], sem0).wait()

    @pl.when(par == 1)
    def _w1():
        pltpu.make_async_copy(acc_hbm.at[0], accv_ref.at[1], sem1).wait()

    acc_p = jnp.dot(
        mq, jnp.squeeze(accv_ref[par, 0], axis=1), preferred_element_type=jnp.float32
    )
    acc_r = jnp.dot(
        mq, jnp.squeeze(accv_ref[par, 1], axis=1), preferred_element_type=jnp.float32
    )

    diff = acc_p / counts_safe - acc_r / rep_safe
    mse = jnp.sum(diff * diff, axis=1, keepdims=True) / _D
    feat_contrib = jnp.sum(has_rep * mse)
    cnt_contrib = jnp.sum(has_rep)

    sacc_ref[0] += feat_contrib
    sacc_ref[1] += cnt_contrib
    sacc_ref[2] += attn_contrib

    @pl.when(bidx == nb - 1)
    def _fin():
        fls = sacc_ref[0]
        fc = sacc_ref[1]
        feat_loss = jnp.where(fc > 0.0, fls / jnp.maximum(fc, 1.0), fls)
        out_ref[0, 0] = _W_FEAT * feat_loss + _W_ATTN * sacc_ref[2] / nb


def kernel(attn, feat):
    B = attn.shape[0]
    attn3 = attn.reshape(B, 1, _T)

    idx_w, aux, gfv = pl.pallas_call(
        _prep_kernel,
        in_specs=[pl.BlockSpec((B, 1, _T), lambda: (0, 0, 0))],
        out_specs=[
            pl.BlockSpec((_NQ, B, 2 * _QT), lambda: (0, 0, 0)),
            pl.BlockSpec((B, _NSEG, 128), lambda: (0, 0, 0)),
            pl.BlockSpec((B, 1, 128), lambda: (0, 0, 0)),
        ],
        out_shape=[
            jax.ShapeDtypeStruct((_NQ, B, 2 * _QT), jnp.int32),
            jax.ShapeDtypeStruct((B, _NSEG, 128), jnp.float32),
            jax.ShapeDtypeStruct((B, 1, 128), jnp.float32),
        ],
    )(attn3)

    feat_flat = feat.reshape(B, _T * _D)
    acc5 = _sc_scatter(feat_flat, idx_w)

    out = pl.pallas_call(
        _finalize_kernel,
        grid=(B,),
        in_specs=[
            pl.BlockSpec((1, _NSEG, 128), lambda b: (b, 0, 0)),
            pl.BlockSpec((1, 1, 128), lambda b: (b, 0, 0)),
            pl.BlockSpec(memory_space=pl.ANY),
        ],
        out_specs=pl.BlockSpec(memory_space=pltpu.SMEM),
        out_shape=jax.ShapeDtypeStruct((1, 1), jnp.float32),
        scratch_shapes=[
            pltpu.SMEM((3,), jnp.float32),
            pltpu.VMEM((2, 2, _NQ * _LSEG, 1, _D), jnp.float32),
            pltpu.SemaphoreType.DMA,
            pltpu.SemaphoreType.DMA,
        ],
    )(aux, gfv, acc5)
    return out[0, 0]
